# baseline (speedup 1.0000x reference)
import functools

import jax
import jax.numpy as jnp
from jax import lax
from jax.experimental import pallas as pl
from jax.experimental.pallas import tpu as pltpu
from jax.experimental.pallas import tpu_sc as plsc

_K_ACTIVE = 5000
_C = 128
_NCORES = 2
_NSUB = 16
_NW = _NCORES * _NSUB


def _linear_mask_body(x_ref, ranks_ref, wt_ref, b_ref, out_ref):
    h = jnp.dot(x_ref[...], wt_ref[...], preferred_element_type=jnp.float32)
    h = h + b_ref[...]
    active = (ranks_ref[...] <= _K_ACTIVE).astype(jnp.float32)
    out_ref[...] = h * active


def _linear_mask(x, ranks_col, wt, b_row):
    n, d = x.shape
    blk = 2000
    grid = n // blk
    return pl.pallas_call(
        _linear_mask_body,
        grid=(grid,),
        in_specs=[
            pl.BlockSpec((blk, d), lambda i: (i, 0)),
            pl.BlockSpec((blk, 1), lambda i: (i, 0)),
            pl.BlockSpec((d, d), lambda i: (0, 0)),
            pl.BlockSpec((1, d), lambda i: (0, 0)),
        ],
        out_specs=pl.BlockSpec((blk, d), lambda i: (i, 0)),
        out_shape=jax.ShapeDtypeStruct((n, d), jnp.float32),
    )(x, ranks_col, wt, b_row)


def _sum_body(a_ref, b_ref, out_ref):
    out_ref[...] = a_ref[...] + b_ref[...]


def _sum_partials(partials, nacc, d):
    blk = nacc // 4
    return pl.pallas_call(
        _sum_body,
        grid=(4,),
        in_specs=[
            pl.BlockSpec((blk, d), lambda j: (j, 0)),
            pl.BlockSpec((blk, d), lambda j: (j + 4, 0)),
        ],
        out_specs=pl.BlockSpec((blk, d), lambda j: (j, 0)),
        out_shape=jax.ShapeDtypeStruct((nacc, d), jnp.float32),
    )(partials, partials)


def _make_sc_aggregate(n, d, nacc, g):
    rows_per_tile = nacc // _NSUB
    half = g // 2
    mesh = plsc.VectorSubcoreMesh(core_axis_name="c", subcore_axis_name="s")

    @functools.partial(
        pl.kernel,
        out_type=jax.ShapeDtypeStruct((_NCORES * nacc, d), jnp.float32),
        mesh=mesh,
        scratch_types=[
            pltpu.VMEM_SHARED((nacc, d), jnp.float32),
            pltpu.VMEM((g, _C), jnp.int32),
            pltpu.VMEM((_C,), jnp.int32),
            pltpu.VMEM((_C,), jnp.int32),
            pltpu.VMEM((_C, d), jnp.float32),
            pltpu.VMEM((_C, d), jnp.float32),
            pltpu.SemaphoreType.DMA,
            pltpu.SemaphoreType.DMA,
            pltpu.SemaphoreType.DMA,
            pltpu.SemaphoreType.DMA,
        ],
    )
    def k(h_hbm, src_hbm, dst_hbm, out_hbm, acc, srcs, dbuf_a, dbuf_b,
          rows_a, rows_b, sem_a, sem_b, sem_da, sem_db):
        cid = lax.axis_index("c")
        sid = lax.axis_index("s")
        wid = cid * _NSUB + sid
        ebase = wid * g * _C

        def zfill(i, carry):
            rows_a[i // 8, pl.ds((i % 8) * 16, 16)] = jnp.zeros(
                (16,), jnp.float32)
            return carry

        lax.fori_loop(0, _C * (d // 16), zfill, 0)

        def zcopy(j, carry):
            pltpu.sync_copy(
                rows_a, acc.at[pl.ds(sid * rows_per_tile + j * _C, _C)])
            return carry

        lax.fori_loop(0, rows_per_tile // _C, zcopy, 0)
        pltpu.sync_copy(src_hbm.at[pl.ds(wid * g, g)], srcs)
        plsc.subcore_barrier()

        pltpu.async_copy(h_hbm.at[srcs.at[0]], rows_a, sem_a)
        pltpu.async_copy(dst_hbm.at[pl.ds(ebase, _C)], dbuf_a, sem_da)

        def step(i, carry):
            c0 = 2 * i
            c1 = c0 + 1
            pltpu.async_copy(h_hbm.at[srcs.at[c1]], rows_b, sem_b)
            pltpu.async_copy(
                dst_hbm.at[pl.ds(ebase + c1 * _C, _C)], dbuf_b, sem_db)
            pltpu.make_async_copy(h_hbm.at[srcs.at[c0]], rows_a, sem_a).wait()
            pltpu.make_async_copy(
                dst_hbm.at[pl.ds(ebase, _C)], dbuf_a, sem_da).wait()
            pltpu.sync_copy(rows_a, acc.at[dbuf_a], add=True)

            @pl.when(i < half - 1)
            def _():
                pltpu.async_copy(h_hbm.at[srcs.at[c0 + 2]], rows_a, sem_a)
                pltpu.async_copy(
                    dst_hbm.at[pl.ds(ebase + (c0 + 2) * _C, _C)],
                    dbuf_a, sem_da)

            pltpu.make_async_copy(h_hbm.at[srcs.at[c1]], rows_b, sem_b).wait()
            pltpu.make_async_copy(
                dst_hbm.at[pl.ds(ebase, _C)], dbuf_b, sem_db).wait()
            pltpu.sync_copy(rows_b, acc.at[dbuf_b], add=True)
            return carry

        lax.fori_loop(0, half, step, 0)
        plsc.subcore_barrier()

        r0 = sid * rows_per_tile
        pltpu.sync_copy(
            acc.at[pl.ds(r0, rows_per_tile)],
            out_hbm.at[pl.ds(cid * nacc + r0, rows_per_tile)])

    return k


def kernel(x, edge_index, node_rankings, W, b):
    n, d = x.shape
    e = edge_index.shape[1]

    h_act = _linear_mask(
        x, node_rankings[0][:, None], W.T, b[None, :])

    g = -(-e // (_NW * _C))
    g = ((g + 7) // 8) * 8
    e_pad = _NW * g * _C
    nacc = ((n + _NSUB * 64 - 1) // (_NSUB * 64)) * (_NSUB * 64)
    src = edge_index[0]
    dst = edge_index[1]
    pad = e_pad - e
    src_p = jnp.concatenate([src, jnp.zeros((pad,), jnp.int32)]).reshape(-1, _C)
    dst_p = jnp.concatenate([dst, jnp.full((pad,), n, jnp.int32)])

    partials = _make_sc_aggregate(n, d, nacc, g)(h_act, src_p, dst_p)
    out = _sum_partials(partials, nacc, d)
    return out[:n]


# speedup vs baseline: 1.0029x; 1.0029x over previous
import functools

import jax
import jax.numpy as jnp
from jax import lax
from jax.experimental import pallas as pl
from jax.experimental.pallas import tpu as pltpu
from jax.experimental.pallas import tpu_sc as plsc

_K_ACTIVE = 5000
_C = 64
_K = 4
_NCORES = 2
_NSUB = 16
_NW = _NCORES * _NSUB


def _linear_mask_body(x_ref, ranks_ref, wt_ref, b_ref, out_ref):
    h = jnp.dot(x_ref[...], wt_ref[...], preferred_element_type=jnp.float32)
    h = h + b_ref[...]
    active = (ranks_ref[...] <= _K_ACTIVE).astype(jnp.float32)
    out_ref[...] = h * active


def _linear_mask(x, ranks_col, wt, b_row):
    n, d = x.shape
    blk = 2000
    grid = n // blk
    return pl.pallas_call(
        _linear_mask_body,
        grid=(grid,),
        in_specs=[
            pl.BlockSpec((blk, d), lambda i: (i, 0)),
            pl.BlockSpec((blk, 1), lambda i: (i, 0)),
            pl.BlockSpec((d, d), lambda i: (0, 0)),
            pl.BlockSpec((1, d), lambda i: (0, 0)),
        ],
        out_specs=pl.BlockSpec((blk, d), lambda i: (i, 0)),
        out_shape=jax.ShapeDtypeStruct((n, d), jnp.float32),
    )(x, ranks_col, wt, b_row)


def _sum_body(a_ref, b_ref, out_ref):
    out_ref[...] = a_ref[...] + b_ref[...]


def _sum_partials(partials, nacc, d):
    blk = nacc // 4
    return pl.pallas_call(
        _sum_body,
        grid=(4,),
        in_specs=[
            pl.BlockSpec((blk, d), lambda j: (j, 0)),
            pl.BlockSpec((blk, d), lambda j: (j + 4, 0)),
        ],
        out_specs=pl.BlockSpec((blk, d), lambda j: (j, 0)),
        out_shape=jax.ShapeDtypeStruct((nacc, d), jnp.float32),
    )(partials, partials)


def _make_sc_aggregate(n, d, nacc, g):
    rows_per_tile = nacc // _NSUB
    sb = 2 * _K
    iters = g // sb
    mesh = plsc.VectorSubcoreMesh(core_axis_name="c", subcore_axis_name="s")

    @functools.partial(
        pl.kernel,
        out_type=jax.ShapeDtypeStruct((_NCORES * nacc, d), jnp.float32),
        mesh=mesh,
        scratch_types=[
            pltpu.VMEM_SHARED((nacc, d), jnp.float32),
            [pltpu.VMEM((_C,), jnp.int32) for _ in range(sb)],
            [pltpu.VMEM((_C,), jnp.int32) for _ in range(_K)],
            [pltpu.VMEM((_C, d), jnp.float32) for _ in range(_K)],
            [pltpu.SemaphoreType.DMA for _ in range(sb)],
            [pltpu.SemaphoreType.DMA for _ in range(_K)],
            [pltpu.SemaphoreType.DMA for _ in range(_K)],
        ],
    )
    def k(h_hbm, src_hbm, dst_hbm, out_hbm, acc, sbufs, dbufs, rows,
          ssems, gsems, dsems):
        cid = lax.axis_index("c")
        sid = lax.axis_index("s")
        wid = cid * _NSUB + sid
        ebase = wid * g * _C

        def swait(j):
            pltpu.make_async_copy(
                src_hbm.at[pl.ds(ebase, _C)], sbufs[j], ssems[j]).wait()

        def zfill(i, carry):
            rows[0][i // (d // 16), pl.ds((i % (d // 16)) * 16, 16)] = (
                jnp.zeros((16,), jnp.float32))
            return carry

        lax.fori_loop(0, _C * (d // 16), zfill, 0)

        def zcopy(j, carry):
            pltpu.sync_copy(
                rows[0], acc.at[pl.ds(sid * rows_per_tile + j * _C, _C)])
            return carry

        lax.fori_loop(0, rows_per_tile // _C, zcopy, 0)

        for j in range(sb):
            pltpu.async_copy(
                src_hbm.at[pl.ds(ebase + j * _C, _C)], sbufs[j], ssems[j])
        for j in range(_K):
            swait(j)
            pltpu.async_copy(
                h_hbm.at[sbufs[j]], rows[j], gsems[j])
            pltpu.async_copy(
                dst_hbm.at[pl.ds(ebase + j * _C, _C)], dbufs[j], dsems[j])
        plsc.subcore_barrier()

        def step(i, carry):
            for j in range(sb):
                c = i * sb + j
                kk = j % _K
                pltpu.make_async_copy(
                    h_hbm.at[sbufs[j]], rows[kk], gsems[kk]).wait()
                pltpu.make_async_copy(
                    dst_hbm.at[pl.ds(ebase, _C)], dbufs[kk], dsems[kk]).wait()
                pltpu.sync_copy(rows[kk], acc.at[dbufs[kk]], add=True)

                @pl.when(c + sb < g)
                def _():
                    pltpu.async_copy(
                        src_hbm.at[pl.ds(ebase + (c + sb) * _C, _C)],
                        sbufs[j], ssems[j])

                @pl.when(c + _K < g)
                def _():
                    swait((j + _K) % sb)
                    pltpu.async_copy(
                        h_hbm.at[sbufs[(j + _K) % sb]], rows[kk], gsems[kk])
                    pltpu.async_copy(
                        dst_hbm.at[pl.ds(ebase + (c + _K) * _C, _C)],
                        dbufs[kk], dsems[kk])
            return carry

        lax.fori_loop(0, iters, step, 0)
        plsc.subcore_barrier()

        r0 = sid * rows_per_tile
        pltpu.sync_copy(
            acc.at[pl.ds(r0, rows_per_tile)],
            out_hbm.at[pl.ds(cid * nacc + r0, rows_per_tile)])

    return k


def kernel(x, edge_index, node_rankings, W, b):
    n, d = x.shape
    e = edge_index.shape[1]

    h_act = _linear_mask(
        x, node_rankings[0][:, None], W.T, b[None, :])

    g = -(-e // (_NW * _C))
    g = ((g + 2 * _K - 1) // (2 * _K)) * (2 * _K)
    e_pad = _NW * g * _C
    nacc = ((n + _NSUB * 64 - 1) // (_NSUB * 64)) * (_NSUB * 64)
    src = edge_index[0]
    dst = edge_index[1]
    pad = e_pad - e
    src_p = jnp.concatenate([src, jnp.zeros((pad,), jnp.int32)])
    dst_p = jnp.concatenate([dst, jnp.full((pad,), n, jnp.int32)])

    partials = _make_sc_aggregate(n, d, nacc, g)(h_act, src_p, dst_p)
    out = _sum_partials(partials, nacc, d)
    return out[:n]
